# baseline (speedup 1.0000x reference)
import functools

import jax
import jax.numpy as jnp
from jax import lax
from jax.experimental import pallas as pl
from jax.experimental.pallas import tpu as pltpu
from jax.experimental.pallas import tpu_sc as plsc

_LANES = 16
_POS_PER_CHUNK = 16


def kernel(x, pe_table):
    B, S, D = x.shape

    info = plsc.get_sparse_core_info()
    NC, NS = info.num_cores, info.num_subcores
    NW = NC * NS
    pos_per_w = S // NW
    P = _POS_PER_CHUNK
    n_chunks = pos_per_w // P

    @functools.partial(
        pl.kernel,
        mesh=plsc.VectorSubcoreMesh(core_axis_name="c", subcore_axis_name="s"),
        out_type=jax.ShapeDtypeStruct((B, S, D), jnp.float32),
        scratch_types=[
            pltpu.VMEM((B, P, D), jnp.float32),
            pltpu.VMEM((B, P, D), jnp.float32),
            pltpu.VMEM((P, D), jnp.float32),
            pltpu.VMEM((P, D), jnp.float32),
            pltpu.SemaphoreType.DMA,
            pltpu.SemaphoreType.DMA,
            pltpu.SemaphoreType.DMA,
            pltpu.SemaphoreType.DMA,
            pltpu.SemaphoreType.DMA,
            pltpu.SemaphoreType.DMA,
        ],
    )
    def sc_add(x_hbm, pe_hbm, out_hbm, xa, xb, pea, peb,
               sem_xa, sem_xb, sem_pea, sem_peb, sem_oa, sem_ob):
        wid = lax.axis_index("s") * NC + lax.axis_index("c")
        base_pos = wid * pos_per_w

        xbufs = (xa, xb)
        pebufs = (pea, peb)
        xsems = (sem_xa, sem_xb)
        pesems = (sem_pea, sem_peb)
        osems = (sem_oa, sem_ob)

        handles = {}

        def pos0(ci):
            return base_pos + ci * P

        def x_in(ci, i):
            return [
                pltpu.async_copy(
                    x_hbm.at[pl.ds(2 * h, 2), pl.ds(pos0(ci), P), :],
                    xbufs[i].at[pl.ds(2 * h, 2)], xsems[i])
                for h in range(2)
            ]

        def x_out(ci, i):
            return [
                pltpu.async_copy(
                    xbufs[i].at[pl.ds(2 * h, 2)],
                    out_hbm.at[pl.ds(2 * h, 2), pl.ds(pos0(ci), P), :],
                    osems[i])
                for h in range(2)
            ]

        def wait_all(hs):
            for h in hs:
                h.wait()

        handles[("x", 0)] = x_in(0, 0)
        handles[("pe", 0)] = pltpu.async_copy(
            pe_hbm.at[pl.ds(pos0(0), P), :], pebufs[0], pesems[0])

        for ci in range(n_chunks):
            xi = ci % 2

            if ci + 1 < n_chunks:
                ni = (ci + 1) % 2
                if ("o", ci - 1) in handles:
                    wait_all(handles[("o", ci - 1)])
                handles[("x", ci + 1)] = x_in(ci + 1, ni)
                handles[("pe", ci + 1)] = pltpu.async_copy(
                    pe_hbm.at[pl.ds(pos0(ci + 1), P), :],
                    pebufs[ni], pesems[ni])

            wait_all(handles[("x", ci)])
            handles[("pe", ci)].wait()

            xbuf = xbufs[xi]
            pebuf = pebufs[xi]

            def row_body(r, carry):
                @plsc.parallel_loop(0, D, step=_LANES, unroll=8)
                def slice_body(c):
                    sl = pl.ds(c, _LANES)
                    pv = pebuf[r, sl]
                    for b in range(B):
                        xbuf[b, r, sl] = xbuf[b, r, sl] + pv

                return carry

            lax.fori_loop(0, P, row_body, 0)

            handles[("o", ci)] = x_out(ci, xi)

        wait_all(handles[("o", n_chunks - 2)])
        wait_all(handles[("o", n_chunks - 1)])

    out = sc_add(x, pe_table)
    return out


# speedup vs baseline: 1.0081x; 1.0081x over previous
import functools

import jax
import jax.numpy as jnp
from jax import lax
from jax.experimental import pallas as pl
from jax.experimental.pallas import tpu as pltpu
from jax.experimental.pallas import tpu_sc as plsc

_LANES = 16
_POS_PER_CHUNK = 16


def kernel(x, pe_table):
    B, S, D = x.shape

    info = plsc.get_sparse_core_info()
    NC, NS = info.num_cores, info.num_subcores
    NW = NC * NS
    pos_per_w = S // NW
    P = _POS_PER_CHUNK
    n_chunks = pos_per_w // P

    @functools.partial(
        pl.kernel,
        mesh=plsc.VectorSubcoreMesh(core_axis_name="c", subcore_axis_name="s"),
        out_type=jax.ShapeDtypeStruct((B, S, D), jnp.float32),
        scratch_types=[
            pltpu.VMEM((B, P, D), jnp.float32),
            pltpu.VMEM((B, P, D), jnp.float32),
            pltpu.VMEM((P, D), jnp.float32),
            pltpu.VMEM((P, D), jnp.float32),
            pltpu.SemaphoreType.DMA,
            pltpu.SemaphoreType.DMA,
            pltpu.SemaphoreType.DMA,
            pltpu.SemaphoreType.DMA,
            pltpu.SemaphoreType.DMA,
            pltpu.SemaphoreType.DMA,
        ],
    )
    def sc_add(x_hbm, pe_hbm, out_hbm, xa, xb, pea, peb,
               sem_xa, sem_xb, sem_pea, sem_peb, sem_oa, sem_ob):
        wid = lax.axis_index("s") * NC + lax.axis_index("c")
        base_pos = wid * pos_per_w

        xbufs = (xa, xb)
        pebufs = (pea, peb)
        xsems = (sem_xa, sem_xb)
        pesems = (sem_pea, sem_peb)
        osems = (sem_oa, sem_ob)

        handles = {}

        def pos0(ci):
            return base_pos + ci * P

        handles[("x", 0)] = pltpu.async_copy(
            x_hbm.at[:, pl.ds(pos0(0), P), :], xbufs[0], xsems[0])
        handles[("pe", 0)] = pltpu.async_copy(
            pe_hbm.at[pl.ds(pos0(0), P), :], pebufs[0], pesems[0])

        for ci in range(n_chunks):
            xi = ci % 2

            if ci + 1 < n_chunks:
                ni = (ci + 1) % 2
                if ("o", ci - 1) in handles:
                    handles[("o", ci - 1)].wait()
                handles[("x", ci + 1)] = pltpu.async_copy(
                    x_hbm.at[:, pl.ds(pos0(ci + 1), P), :],
                    xbufs[ni], xsems[ni])
                handles[("pe", ci + 1)] = pltpu.async_copy(
                    pe_hbm.at[pl.ds(pos0(ci + 1), P), :],
                    pebufs[ni], pesems[ni])

            handles[("x", ci)].wait()
            handles[("pe", ci)].wait()

            xbuf = xbufs[xi]
            pebuf = pebufs[xi]

            def row_body(r, carry):
                @plsc.parallel_loop(0, D, step=_LANES, unroll=8)
                def slice_body(c):
                    sl = pl.ds(c, _LANES)
                    pv = pebuf[r, sl]
                    for b in range(B):
                        xbuf[b, r, sl] = xbuf[b, r, sl] + pv

                return carry

            lax.fori_loop(0, P, row_body, 0)

            handles[("o", ci)] = pltpu.async_copy(
                xbuf, out_hbm.at[:, pl.ds(pos0(ci), P), :], osems[xi])

        handles[("o", n_chunks - 2)].wait()
        handles[("o", n_chunks - 1)].wait()

    out = sc_add(x, pe_table)
    return out
